# baseline (speedup 1.0000x reference)
import functools

import numpy as np
import jax
import jax.numpy as jnp
from jax import lax
from jax.experimental import pallas as pl
from jax.experimental.pallas import tpu as pltpu
from jax.experimental.pallas import tpu_sc as plsc

BATCH = 4096
NUM_DENSE = 13
NUM_FIELDS = 26
VOCAB = 100000
EMBED_DIM = 32
LN_BOT = 64
LN_TOP = 256
EMB_FLAT = NUM_FIELDS * EMBED_DIM

NC, NS, L = 2, 16, 16
NW = NC * NS
NB = BATCH // NW
IDX_PER_W = NB * NUM_FIELDS
VSUB = 12544
VPAD = 8 * VSUB


def _fold_body(dense_ref, wbot_ref, bbot_ref, wteT_ref, wtdT_ref, btop_ref,
               wemb_ref, dp_ref):
    wemb_ref[...] = jnp.mean(wteT_ref[...], axis=0, keepdims=True)
    wd = jnp.mean(wtdT_ref[...], axis=0, keepdims=True)
    vb = lax.dot_general(wbot_ref[...], wd, (((1,), (1,)), ((), ())))
    c = jnp.sum(bbot_ref[...][None, :] * wd) + jnp.mean(btop_ref[...])
    dp = lax.dot_general(dense_ref[...], vb, (((1,), (0,)), ((), ())))
    dp_ref[...] = dp + c


def _fold(dense, W_bot, b_bot, wteT, wtdT, b_top):
    return pl.pallas_call(
        _fold_body,
        out_shape=(
            jax.ShapeDtypeStruct((1, EMB_FLAT), jnp.float32),
            jax.ShapeDtypeStruct((BATCH, 1), jnp.float32),
        ),
    )(dense, W_bot, b_bot, wteT, wtdT, b_top)


def _collapse_body(t_ref, w_ref, u_ref):
    for s in range(8):
        u_ref[pl.ds(s, 1), :] = lax.dot_general(
            w_ref[0], t_ref[0, :, pl.ds(s * VSUB, VSUB)],
            (((1,), (0,)), ((), ())), preferred_element_type=jnp.float32)


def _collapse(tablesT, w26):
    return pl.pallas_call(
        _collapse_body,
        grid=(NUM_FIELDS,),
        in_specs=[
            pl.BlockSpec((1, EMBED_DIM, VPAD), lambda f: (f, 0, 0)),
            pl.BlockSpec((1, 1, EMBED_DIM), lambda f: (f, 0, 0)),
        ],
        out_specs=pl.BlockSpec((8, VSUB), lambda f: (f, 0)),
        out_shape=jax.ShapeDtypeStruct((NUM_FIELDS * 8, VSUB), jnp.float32),
    )(tablesT, w26)


_MESH = plsc.VectorSubcoreMesh(core_axis_name="c", subcore_axis_name="s")


@functools.partial(
    pl.kernel,
    out_type=jax.ShapeDtypeStruct((BATCH,), jnp.float32),
    mesh=_MESH,
    compiler_params=pltpu.CompilerParams(needs_layout_passes=False,
                                         use_tc_tiling_on_sc=False),
    scratch_types=[
        pltpu.VMEM((IDX_PER_W,), jnp.int32),
        pltpu.VMEM((IDX_PER_W,), jnp.int32),
        pltpu.VMEM((IDX_PER_W,), jnp.float32),
        pltpu.VMEM((NB,), jnp.float32),
        pltpu.VMEM((NB,), jnp.float32),
        pltpu.SemaphoreType.DMA,
        pltpu.SemaphoreType.DMA,
    ],
)
def _sc_dlrm(idxT_hbm, off_hbm, u_hbm, dp_hbm, out_hbm,
             idx_v, off_v, g_v, dp_v, out_v, sem_i, sem_g):
    wid = lax.axis_index("s") * NC + lax.axis_index("c")
    base_b = pl.multiple_of(wid * NB, 8)

    icopies = [
        pltpu.async_copy(
            idxT_hbm.at[pl.ds(pl.multiple_of(f * BATCH + wid * NB, 8), NB)],
            idx_v.at[pl.ds(f * NB, NB)],
            sem_i,
        )
        for f in range(NUM_FIELDS)
    ]
    pltpu.sync_copy(off_hbm, off_v)
    pltpu.sync_copy(dp_hbm.at[pl.ds(base_b, NB)], dp_v)
    for cp in icopies:
        cp.wait()

    def _addoff(t, carry):
        o = pl.multiple_of(t * L, 8)
        idx_v[pl.ds(o, L)] = idx_v[pl.ds(o, L)] + off_v[pl.ds(o, L)]
        return carry

    lax.fori_loop(0, IDX_PER_W // L, _addoff, 0)

    gcopies = [
        pltpu.async_copy(
            u_hbm.at[idx_v.at[pl.ds(f * NB, NB)]],
            g_v.at[pl.ds(f * NB, NB)],
            sem_g,
        )
        for f in range(NUM_FIELDS)
    ]
    for cp in gcopies:
        cp.wait()

    for j in range(NB // L):
        acc = dp_v[pl.ds(j * L, L)]
        for f in range(NUM_FIELDS):
            acc = acc + g_v[pl.ds(f * NB + j * L, L)]
        out_v[pl.ds(j * L, L)] = 1.0 / (1.0 + jnp.exp(-acc))
    pltpu.sync_copy(out_v, out_hbm.at[pl.ds(base_b, NB)])


_OFFSETS = np.repeat(np.arange(NUM_FIELDS, dtype=np.int32) * VPAD, NB)


def kernel(dense_features, sparse_features, tables, W_bot, b_bot, W_top, b_top):
    wteT = W_top[:EMB_FLAT, :].T
    wtdT = W_top[EMB_FLAT:, :].T
    wemb, dp = _fold(dense_features, W_bot, b_bot, wteT, wtdT, b_top)
    tablesT = jnp.transpose(tables, (0, 2, 1))
    u = _collapse(tablesT, wemb.reshape(NUM_FIELDS, 1, EMBED_DIM))
    out = _sc_dlrm(
        jnp.transpose(sparse_features).reshape(-1),
        jnp.asarray(_OFFSETS),
        u.reshape(-1),
        dp.reshape(-1),
    )
    return out


# speedup vs baseline: 1.0051x; 1.0051x over previous
import functools

import numpy as np
import jax
import jax.numpy as jnp
from jax import lax
from jax.experimental import pallas as pl
from jax.experimental.pallas import tpu as pltpu
from jax.experimental.pallas import tpu_sc as plsc

BATCH = 4096
NUM_DENSE = 13
NUM_FIELDS = 26
VOCAB = 100000
EMBED_DIM = 32
LN_BOT = 64
LN_TOP = 256
EMB_FLAT = NUM_FIELDS * EMBED_DIM

NC, NS, L = 2, 16, 16
NW = NC * NS
NB = BATCH // NW
IDX_PER_W = NB * NUM_FIELDS
VSUB = 12544
VPAD = 8 * VSUB


def _fold_body(dense_ref, wbot_ref, bbot_ref, wteT_ref, wtdT_ref, btop_ref,
               wemb_ref, dp_ref):
    wemb_ref[...] = jnp.mean(wteT_ref[...], axis=0, keepdims=True)
    wd = jnp.mean(wtdT_ref[...], axis=0, keepdims=True)
    vb = lax.dot_general(wbot_ref[...], wd, (((1,), (1,)), ((), ())))
    c = jnp.sum(bbot_ref[...][None, :] * wd) + jnp.mean(btop_ref[...])
    dp = lax.dot_general(dense_ref[...], vb, (((1,), (0,)), ((), ())))
    dp_ref[...] = dp + c


def _fold(dense, W_bot, b_bot, wteT, wtdT, b_top):
    return pl.pallas_call(
        _fold_body,
        out_shape=(
            jax.ShapeDtypeStruct((1, EMB_FLAT), jnp.float32),
            jax.ShapeDtypeStruct((BATCH, 1), jnp.float32),
        ),
    )(dense, W_bot, b_bot, wteT, wtdT, b_top)


def _collapse_body(t_ref, w_ref, u_ref):
    for s in range(8):
        u_ref[pl.ds(s, 1), :] = lax.dot_general(
            w_ref[0], t_ref[0, :, pl.ds(s * VSUB, VSUB)],
            (((1,), (0,)), ((), ())), preferred_element_type=jnp.float32)


def _collapse(tablesT, w26):
    return pl.pallas_call(
        _collapse_body,
        grid=(NUM_FIELDS,),
        in_specs=[
            pl.BlockSpec((1, EMBED_DIM, VPAD), lambda f: (f, 0, 0)),
            pl.BlockSpec((1, 1, EMBED_DIM), lambda f: (f, 0, 0)),
        ],
        out_specs=pl.BlockSpec((8, VSUB), lambda f: (f, 0)),
        out_shape=jax.ShapeDtypeStruct((NUM_FIELDS * 8, VSUB), jnp.float32),
        compiler_params=pltpu.CompilerParams(
            dimension_semantics=("parallel",)),
    )(tablesT, w26)


_MESH = plsc.VectorSubcoreMesh(core_axis_name="c", subcore_axis_name="s")


@functools.partial(
    pl.kernel,
    out_type=jax.ShapeDtypeStruct((BATCH,), jnp.float32),
    mesh=_MESH,
    compiler_params=pltpu.CompilerParams(needs_layout_passes=False,
                                         use_tc_tiling_on_sc=False),
    scratch_types=[
        pltpu.VMEM((IDX_PER_W,), jnp.int32),
        pltpu.VMEM((IDX_PER_W,), jnp.int32),
        pltpu.VMEM((IDX_PER_W,), jnp.float32),
        pltpu.VMEM((NB,), jnp.float32),
        pltpu.VMEM((NB,), jnp.float32),
        pltpu.SemaphoreType.DMA,
        pltpu.SemaphoreType.DMA,
    ],
)
def _sc_dlrm(idxT_hbm, off_hbm, u_hbm, dp_hbm, out_hbm,
             idx_v, off_v, g_v, dp_v, out_v, sem_i, sem_g):
    wid = lax.axis_index("s") * NC + lax.axis_index("c")
    base_b = pl.multiple_of(wid * NB, 8)

    icopies = [
        pltpu.async_copy(
            idxT_hbm.at[pl.ds(pl.multiple_of(f * BATCH + wid * NB, 8), NB)],
            idx_v.at[pl.ds(f * NB, NB)],
            sem_i,
        )
        for f in range(NUM_FIELDS)
    ]
    pltpu.sync_copy(off_hbm, off_v)
    pltpu.sync_copy(dp_hbm.at[pl.ds(base_b, NB)], dp_v)
    for cp in icopies:
        cp.wait()

    def _addoff(t, carry):
        o = pl.multiple_of(t * L, 8)
        idx_v[pl.ds(o, L)] = idx_v[pl.ds(o, L)] + off_v[pl.ds(o, L)]
        return carry

    lax.fori_loop(0, IDX_PER_W // L, _addoff, 0)

    gcopies = [
        pltpu.async_copy(
            u_hbm.at[idx_v.at[pl.ds(f * NB, NB)]],
            g_v.at[pl.ds(f * NB, NB)],
            sem_g,
        )
        for f in range(NUM_FIELDS)
    ]
    for cp in gcopies:
        cp.wait()

    for j in range(NB // L):
        acc = dp_v[pl.ds(j * L, L)]
        for f in range(NUM_FIELDS):
            acc = acc + g_v[pl.ds(f * NB + j * L, L)]
        out_v[pl.ds(j * L, L)] = 1.0 / (1.0 + jnp.exp(-acc))
    pltpu.sync_copy(out_v, out_hbm.at[pl.ds(base_b, NB)])


_OFFSETS = np.repeat(np.arange(NUM_FIELDS, dtype=np.int32) * VPAD, NB)


def kernel(dense_features, sparse_features, tables, W_bot, b_bot, W_top, b_top):
    wteT = W_top[:EMB_FLAT, :].T
    wtdT = W_top[EMB_FLAT:, :].T
    wemb, dp = _fold(dense_features, W_bot, b_bot, wteT, wtdT, b_top)
    tablesT = jnp.transpose(tables, (0, 2, 1))
    u = _collapse(tablesT, wemb.reshape(NUM_FIELDS, 1, EMBED_DIM))
    out = _sc_dlrm(
        jnp.transpose(sparse_features).reshape(-1),
        jnp.asarray(_OFFSETS),
        u.reshape(-1),
        dp.reshape(-1),
    )
    return out
